# baseline (speedup 1.0000x reference)
import functools
import jax
import jax.numpy as jnp
from jax import lax
from jax.experimental import pallas as pl
from jax.experimental.pallas import tpu as pltpu
from jax.experimental.pallas import tpu_sc as plsc

N = 10000
E = 320000
D = 128
H1 = 8
O1 = 64

NC = 2
NS = 16
EPT = E // NS
B = 800
G = 32

NT = 10240
HALF = NT // NC
W1SRC = 640
W2SRC = 256
NEG = -1e30
PMASK = 1023


def _take16(x, idx):
    dnums = lax.GatherDimensionNumbers(
        offset_dims=(), collapsed_slice_dims=(0,), start_index_map=(0,))
    return lax.gather(x, idx[:, None], dnums, (1,),
                      mode=lax.GatherScatterMode.PROMISE_IN_BOUNDS)


def _make_edge_pass(F, HN, CH):
    W = F + 128
    NB = W // 128
    PHV = F // HN // 16
    NCHUNK = HALF // CH
    RPT = CH // NS
    mesh = plsc.VectorSubcoreMesh(core_axis_name="c", subcore_axis_name="s",
                                  num_cores=NC, num_subcores=NS)

    @functools.partial(
        pl.kernel,
        out_type=[jax.ShapeDtypeStruct((NT, 128), jnp.float32)
                  for _ in range(NB)],
        mesh=mesh,
        scratch_types=(
            [pltpu.VMEM((B,), jnp.int32),
             pltpu.VMEM((B,), jnp.int32),
             pltpu.VMEM((G,), jnp.int32),
             pltpu.VMEM((G,), jnp.int32),
             pltpu.VMEM((G,), jnp.int32),
             pltpu.VMEM((G, W), jnp.float32),
             pltpu.VMEM((G, 128), jnp.float32),
             pltpu.VMEM((16, 128), jnp.float32)]
            + [pltpu.VMEM((G, 128), jnp.float32) for _ in range(NB)]
            + [pltpu.VMEM_SHARED((CH, 128), jnp.float32)
               for _ in range(NB)]
            + [pltpu.SemaphoreType.DMA,
               pltpu.SemaphoreType.DMA]),
    )
    def edge_pass(src_h, dst_h, ts_h, td_h, zeros_h, *rest):
        out_h = rest[:NB]
        (esrc, edst, g_src, g_dst, g_rel, src_b, dst_b, zbuf) = \
            rest[NB:NB + 8]
        msg_b = rest[NB + 8:NB + 8 + NB]
        acc = rest[NB + 8 + NB:NB + 8 + 2 * NB]
        sem1, sem2 = rest[NB + 8 + 2 * NB:]
        c = lax.axis_index("c")
        s = lax.axis_index("s")
        tbase = s * EPT
        lane = lax.iota(jnp.int32, 16)
        headmask = jnp.minimum(jnp.maximum(HN - lane, 0), 1).astype(
            jnp.float32)
        zv = jnp.zeros((16,), jnp.float32)

        pltpu.sync_copy(zeros_h, zbuf)

        def zrow(i, _):
            for t in range(112 // 16):
                msg_b[NB - 1][i, pl.ds(16 + t * 16, 16)] = zv
            return 0

        lax.fori_loop(0, G, zrow, 0)

        for k in range(NCHUNK):
            clo = c * HALF + k * CH

            def zacc(t, _):
                for b in range(NB):
                    pltpu.sync_copy(
                        zbuf, acc[b].at[pl.ds(s * RPT + t * 16, 16)])
                return 0

            lax.fori_loop(0, RPT // 16, zacc, 0)
            plsc.subcore_barrier()

            def blk(sb, _):
                pltpu.sync_copy(src_h.at[pl.ds(tbase + sb * B, B)], esrc)
                pltpu.sync_copy(dst_h.at[pl.ds(tbase + sb * B, B)], edst)

                def grp(g, _):
                    for k2 in range(G // 16):
                        sv = esrc[pl.ds(g * G + k2 * 16, 16)]
                        dv = edst[pl.ds(g * G + k2 * 16, 16)]
                        rel = dv - clo
                        mi = 1 - lax.shift_right_logical(
                            rel | (CH - 1 - rel), 31)
                        g_src[pl.ds(k2 * 16, 16)] = sv
                        g_dst[pl.ds(k2 * 16, 16)] = mi * dv + (1 - mi) * N
                        g_rel[pl.ds(k2 * 16, 16)] = (
                            mi * rel + (1 - mi) * (rel & PMASK))
                    cp1 = pltpu.async_copy(ts_h.at[g_src], src_b, sem1)
                    cp2 = pltpu.async_copy(td_h.at[g_dst], dst_b, sem2)
                    cp1.wait()
                    cp2.wait()

                    def row(i, _):
                        l = src_b[i, pl.ds(F, 16)] + dst_b[i, pl.ds(0, 16)]
                        l = jnp.maximum(l, 0.0) + 0.2 * jnp.minimum(l, 0.0)
                        ex = jnp.exp(l) * headmask
                        msg_b[NB - 1][i, pl.ds(0, 16)] = ex
                        for j in range(HN):
                            e_j = _take16(
                                ex, jnp.full((16,), j, jnp.int32))
                            for r in range(PHV):
                                col = (j * PHV + r) * 16
                                msg_b[col // 128][i, pl.ds(col % 128, 16)] \
                                    = src_b[i, pl.ds(col, 16)] * e_j
                        return 0

                    lax.fori_loop(0, G, row, 0)
                    for b in range(NB):
                        pltpu.sync_copy(msg_b[b], acc[b].at[g_rel],
                                        add=True)
                    return 0

                lax.fori_loop(0, B // G, grp, 0)
                return 0

            lax.fori_loop(0, EPT // B, blk, 0)
            plsc.subcore_barrier()

            for b in range(NB):
                pltpu.sync_copy(acc[b].at[pl.ds(s * RPT, RPT)],
                                out_h[b].at[pl.ds(clo + s * RPT, RPT)])
            plsc.subcore_barrier()

    return edge_pass


def _tc1_body(x_ref, g_ref, b_ref, ws_ref, wd_ref, ts_ref, td_ref):
    x = x_ref[...]
    mu = jnp.mean(x, axis=-1, keepdims=True)
    var = jnp.mean((x - mu) ** 2, axis=-1, keepdims=True)
    xn = (x - mu) * lax.rsqrt(var + 1e-5) * g_ref[...] + b_ref[...]
    ts_ref[...] = jnp.dot(xn, ws_ref[...],
                          preferred_element_type=jnp.float32)
    td_ref[...] = jnp.dot(xn, wd_ref[...],
                          preferred_element_type=jnp.float32)


def _dense1(x, gamma, beta, W1ext, Wd1):
    blk = 1000
    return pl.pallas_call(
        _tc1_body,
        grid=(N // blk,),
        in_specs=[
            pl.BlockSpec((blk, D), lambda i: (i, 0)),
            pl.BlockSpec((1, D), lambda i: (0, 0)),
            pl.BlockSpec((1, D), lambda i: (0, 0)),
            pl.BlockSpec((D, W1SRC), lambda i: (0, 0)),
            pl.BlockSpec((D, 128), lambda i: (0, 0)),
        ],
        out_specs=[
            pl.BlockSpec((blk, W1SRC), lambda i: (i, 0)),
            pl.BlockSpec((blk, 128), lambda i: (i, 0)),
        ],
        out_shape=[
            jax.ShapeDtypeStruct((N, W1SRC), jnp.float32),
            jax.ShapeDtypeStruct((N, 128), jnp.float32),
        ],
    )(x, gamma.reshape(1, D), beta.reshape(1, D), W1ext, Wd1)


def _tc2_body(r0_ref, r1_ref, r2_ref, r3_ref, re_ref, ts_ref, td_ref,
              b1_ref, r8_ref, ws_ref, wd_ref, ts2_ref, td2_ref):
    rawh = jnp.concatenate(
        [r0_ref[...], r1_ref[...], r2_ref[...], r3_ref[...]], axis=-1)
    ts = ts_ref[...]
    s8 = re_ref[:, 0:H1]
    l = ts[:, H1 * O1:H1 * O1 + H1] + td_ref[:, 0:H1]
    l = jnp.where(l > 0, l, 0.2 * l)
    ex_self = jnp.exp(l)
    den = s8 + ex_self + 1e-16
    r8 = r8_ref[...]
    bcast_ex = jnp.dot(ex_self, r8, preferred_element_type=jnp.float32)
    bcast_rd = jnp.dot(1.0 / den, r8, preferred_element_type=jnp.float32)
    out1 = (rawh + bcast_ex * ts[:, 0:H1 * O1]) * bcast_rd
    t = out1 + b1_ref[...]
    h2 = jnp.where(t > 0, t, jnp.exp(jnp.minimum(t, 0.0)) - 1.0)
    ts2_ref[...] = jnp.dot(h2, ws_ref[...],
                           preferred_element_type=jnp.float32)
    td2_ref[...] = jnp.dot(h2, wd_ref[...],
                           preferred_element_type=jnp.float32)


def _dense2(raw1b, ts1, td1, b1, R8, W2ext, Wd2):
    blk = 1000
    return pl.pallas_call(
        _tc2_body,
        grid=(N // blk,),
        in_specs=[
            pl.BlockSpec((blk, 128), lambda i: (i, 0)),
            pl.BlockSpec((blk, 128), lambda i: (i, 0)),
            pl.BlockSpec((blk, 128), lambda i: (i, 0)),
            pl.BlockSpec((blk, 128), lambda i: (i, 0)),
            pl.BlockSpec((blk, 128), lambda i: (i, 0)),
            pl.BlockSpec((blk, W1SRC), lambda i: (i, 0)),
            pl.BlockSpec((blk, 128), lambda i: (i, 0)),
            pl.BlockSpec((1, H1 * O1), lambda i: (0, 0)),
            pl.BlockSpec((H1, H1 * O1), lambda i: (0, 0)),
            pl.BlockSpec((H1 * O1, W2SRC), lambda i: (0, 0)),
            pl.BlockSpec((H1 * O1, 128), lambda i: (0, 0)),
        ],
        out_specs=[
            pl.BlockSpec((blk, W2SRC), lambda i: (i, 0)),
            pl.BlockSpec((blk, 128), lambda i: (i, 0)),
        ],
        out_shape=[
            jax.ShapeDtypeStruct((N, W2SRC), jnp.float32),
            jax.ShapeDtypeStruct((N, 128), jnp.float32),
        ],
    )(raw1b[0], raw1b[1], raw1b[2], raw1b[3], raw1b[4], ts1, td1,
      b1.reshape(1, H1 * O1), R8, W2ext, Wd2)


def _tc3_body(rh_ref, re_ref, ts_ref, td_ref, b2_ref, out_ref):
    ts = ts_ref[...]
    l = ts[:, D:D + 1] + td_ref[:, 0:1]
    l = jnp.where(l > 0, l, 0.2 * l)
    ex = jnp.exp(l)
    den = re_ref[:, 0:1] + ex + 1e-16
    out_ref[...] = ((rh_ref[...] + ex * ts[:, 0:D]) * (1.0 / den)
                    + b2_ref[...])


def _dense3(raw2b, ts2, td2, b2):
    blk = 1000
    return pl.pallas_call(
        _tc3_body,
        grid=(N // blk,),
        in_specs=[
            pl.BlockSpec((blk, 128), lambda i: (i, 0)),
            pl.BlockSpec((blk, 128), lambda i: (i, 0)),
            pl.BlockSpec((blk, W2SRC), lambda i: (i, 0)),
            pl.BlockSpec((blk, 128), lambda i: (i, 0)),
            pl.BlockSpec((1, D), lambda i: (0, 0)),
        ],
        out_specs=pl.BlockSpec((blk, D), lambda i: (i, 0)),
        out_shape=jax.ShapeDtypeStruct((N, D), jnp.float32),
    )(raw2b[0], raw2b[1], ts2, td2, b2.reshape(1, D))


def _head_projector(att, heads, odim):
    eye = jnp.eye(heads, dtype=jnp.float32)
    A = (eye[:, None, :] * att[:, :, None]).reshape(heads * odim, heads)
    return jnp.concatenate(
        [A, jnp.zeros((heads * odim, 16 - heads), jnp.float32)], axis=1)


def kernel(x, edge_index, gamma, beta, W1, att_src1, att_dst1, b1, W2,
           att_src2, att_dst2, b2):
    src = edge_index[0].astype(jnp.int32)
    dst = edge_index[1].astype(jnp.int32)

    A_s1 = _head_projector(att_src1, H1, O1)
    A_d1 = _head_projector(att_dst1, H1, O1)
    W1ext = jnp.concatenate(
        [W1, W1 @ A_s1, jnp.zeros((D, W1SRC - H1 * O1 - 16), jnp.float32)],
        axis=1)
    Wd1 = jnp.concatenate(
        [W1 @ A_d1, jnp.zeros((D, 112), jnp.float32)], axis=1)
    A_s2 = _head_projector(att_src2, 1, D)
    A_d2 = _head_projector(att_dst2, 1, D)
    W2ext = jnp.concatenate(
        [W2, W2 @ A_s2, jnp.zeros((H1 * O1, W2SRC - D - 16), jnp.float32)],
        axis=1)
    Wd2 = jnp.concatenate(
        [W2 @ A_d2, jnp.zeros((H1 * O1, 112), jnp.float32)], axis=1)
    R8 = (jnp.eye(H1, dtype=jnp.float32)[:, None, :]
          * jnp.ones((H1, O1, 1), jnp.float32)).reshape(H1 * O1, H1).T

    ep1 = _make_edge_pass(H1 * O1, H1, HALF // 4)
    ep2 = _make_edge_pass(D, 1, HALF // 2)

    ts1, td1 = _dense1(x, gamma, beta, W1ext, Wd1)
    td1p = jnp.concatenate(
        [td1, jnp.full((1, 128), NEG, jnp.float32)], axis=0)
    zeros = jnp.zeros((16, 128), jnp.float32)
    raw1b = [r[:N] for r in ep1(src, dst, ts1, td1p, zeros)]

    ts2, td2 = _dense2(raw1b, ts1, td1, b1, R8, W2ext, Wd2)
    td2p = jnp.concatenate(
        [td2, jnp.full((1, 128), NEG, jnp.float32)], axis=0)
    raw2b = [r[:N] for r in ep2(src, dst, ts2, td2p, zeros)]

    return _dense3(raw2b, ts2, td2, b2)


# speedup vs baseline: 1.0005x; 1.0005x over previous
import functools
import jax
import jax.numpy as jnp
from jax import lax
from jax.experimental import pallas as pl
from jax.experimental.pallas import tpu as pltpu
from jax.experimental.pallas import tpu_sc as plsc

N = 10000
E = 320000
D = 128
H1 = 8
O1 = 64

NC = 2
NS = 16
EPT = E // NS
B = 800
G = 32

NT = 10240
HALF = NT // NC
W1SRC = 640
W2SRC = 256
NEG = -1e30
PMASK = 1023


def _take16(x, idx):
    dnums = lax.GatherDimensionNumbers(
        offset_dims=(), collapsed_slice_dims=(0,), start_index_map=(0,))
    return lax.gather(x, idx[:, None], dnums, (1,),
                      mode=lax.GatherScatterMode.PROMISE_IN_BOUNDS)


def _make_edge_pass(F, HN, CH):
    W = F + 128
    NB = W // 128
    PHV = F // HN // 16
    NCHUNK = HALF // CH
    RPT = CH // NS
    mesh = plsc.VectorSubcoreMesh(core_axis_name="c", subcore_axis_name="s",
                                  num_cores=NC, num_subcores=NS)

    @functools.partial(
        pl.kernel,
        out_type=[jax.ShapeDtypeStruct((NT, 128), jnp.float32)
                  for _ in range(NB)],
        mesh=mesh,
        scratch_types=(
            [pltpu.VMEM((B,), jnp.int32),
             pltpu.VMEM((B,), jnp.int32),
             pltpu.VMEM((G,), jnp.int32),
             pltpu.VMEM((G,), jnp.int32),
             pltpu.VMEM((G,), jnp.int32),
             pltpu.VMEM((G, W), jnp.float32),
             pltpu.VMEM((G, 128), jnp.float32),
             pltpu.VMEM((16, 128), jnp.float32)]
            + [pltpu.VMEM((G, 128), jnp.float32) for _ in range(NB)]
            + [pltpu.VMEM_SHARED((CH, 128), jnp.float32)
               for _ in range(NB)]
            + [pltpu.SemaphoreType.DMA,
               pltpu.SemaphoreType.DMA,
               pltpu.SemaphoreType.DMA]),
    )
    def edge_pass(src_h, dst_h, ts_h, td_h, zeros_h, *rest):
        out_h = rest[:NB]
        (esrc, edst, g_src, g_dst, g_rel, src_b, dst_b, zbuf) = \
            rest[NB:NB + 8]
        msg_b = rest[NB + 8:NB + 8 + NB]
        acc = rest[NB + 8 + NB:NB + 8 + 2 * NB]
        sem1, sem2, sem3 = rest[NB + 8 + 2 * NB:]
        c = lax.axis_index("c")
        s = lax.axis_index("s")
        tbase = s * EPT
        lane = lax.iota(jnp.int32, 16)
        headmask = jnp.minimum(jnp.maximum(HN - lane, 0), 1).astype(
            jnp.float32)
        zv = jnp.zeros((16,), jnp.float32)

        pltpu.sync_copy(zeros_h, zbuf)

        def zrow(i, _):
            for t in range(112 // 16):
                msg_b[NB - 1][i, pl.ds(16 + t * 16, 16)] = zv
            return 0

        lax.fori_loop(0, G, zrow, 0)

        for k in range(NCHUNK):
            clo = c * HALF + k * CH

            def zacc(t, _):
                for b in range(NB):
                    pltpu.sync_copy(
                        zbuf, acc[b].at[pl.ds(s * RPT + t * 16, 16)])
                return 0

            lax.fori_loop(0, RPT // 16, zacc, 0)
            plsc.subcore_barrier()

            def blk(sb, _):
                pltpu.sync_copy(src_h.at[pl.ds(tbase + sb * B, B)], esrc)
                pltpu.sync_copy(dst_h.at[pl.ds(tbase + sb * B, B)], edst)

                def grp(g, _):
                    for k2 in range(G // 16):
                        sv = esrc[pl.ds(g * G + k2 * 16, 16)]
                        dv = edst[pl.ds(g * G + k2 * 16, 16)]
                        rel = dv - clo
                        mi = 1 - lax.shift_right_logical(
                            rel | (CH - 1 - rel), 31)
                        g_src[pl.ds(k2 * 16, 16)] = sv
                        g_dst[pl.ds(k2 * 16, 16)] = mi * dv + (1 - mi) * N
                        g_rel[pl.ds(k2 * 16, 16)] = (
                            mi * rel + (1 - mi) * (rel & PMASK))
                    cp1 = pltpu.async_copy(ts_h.at[g_src], src_b, sem1)
                    cp2 = pltpu.async_copy(td_h.at[g_dst], dst_b, sem2)
                    cp1.wait()
                    cp2.wait()

                    def row(i, _):
                        l = src_b[i, pl.ds(F, 16)] + dst_b[i, pl.ds(0, 16)]
                        l = jnp.maximum(l, 0.0) + 0.2 * jnp.minimum(l, 0.0)
                        ex = jnp.exp(l) * headmask
                        msg_b[NB - 1][i, pl.ds(0, 16)] = ex
                        for j in range(HN):
                            e_j = _take16(
                                ex, jnp.full((16,), j, jnp.int32))
                            for r in range(PHV):
                                col = (j * PHV + r) * 16
                                msg_b[col // 128][i, pl.ds(col % 128, 16)] \
                                    = src_b[i, pl.ds(col, 16)] * e_j
                        return 0

                    lax.fori_loop(0, G, row, 0)
                    cps = [pltpu.async_copy(msg_b[b], acc[b].at[g_rel],
                                            sem3, add=True)
                           for b in range(NB)]
                    for cp in cps:
                        cp.wait()
                    return 0

                lax.fori_loop(0, B // G, grp, 0)
                return 0

            lax.fori_loop(0, EPT // B, blk, 0)
            plsc.subcore_barrier()

            for b in range(NB):
                pltpu.sync_copy(acc[b].at[pl.ds(s * RPT, RPT)],
                                out_h[b].at[pl.ds(clo + s * RPT, RPT)])
            plsc.subcore_barrier()

    return edge_pass


def _tc1_body(x_ref, g_ref, b_ref, ws_ref, wd_ref, ts_ref, td_ref):
    x = x_ref[...]
    mu = jnp.mean(x, axis=-1, keepdims=True)
    var = jnp.mean((x - mu) ** 2, axis=-1, keepdims=True)
    xn = (x - mu) * lax.rsqrt(var + 1e-5) * g_ref[...] + b_ref[...]
    ts_ref[...] = jnp.dot(xn, ws_ref[...],
                          preferred_element_type=jnp.float32)
    td_ref[...] = jnp.dot(xn, wd_ref[...],
                          preferred_element_type=jnp.float32)


def _dense1(x, gamma, beta, W1ext, Wd1):
    blk = 1000
    return pl.pallas_call(
        _tc1_body,
        grid=(N // blk,),
        in_specs=[
            pl.BlockSpec((blk, D), lambda i: (i, 0)),
            pl.BlockSpec((1, D), lambda i: (0, 0)),
            pl.BlockSpec((1, D), lambda i: (0, 0)),
            pl.BlockSpec((D, W1SRC), lambda i: (0, 0)),
            pl.BlockSpec((D, 128), lambda i: (0, 0)),
        ],
        out_specs=[
            pl.BlockSpec((blk, W1SRC), lambda i: (i, 0)),
            pl.BlockSpec((blk, 128), lambda i: (i, 0)),
        ],
        out_shape=[
            jax.ShapeDtypeStruct((N, W1SRC), jnp.float32),
            jax.ShapeDtypeStruct((N, 128), jnp.float32),
        ],
    )(x, gamma.reshape(1, D), beta.reshape(1, D), W1ext, Wd1)


def _tc2_body(r0_ref, r1_ref, r2_ref, r3_ref, re_ref, ts_ref, td_ref,
              b1_ref, r8_ref, ws_ref, wd_ref, ts2_ref, td2_ref):
    rawh = jnp.concatenate(
        [r0_ref[...], r1_ref[...], r2_ref[...], r3_ref[...]], axis=-1)
    ts = ts_ref[...]
    s8 = re_ref[:, 0:H1]
    l = ts[:, H1 * O1:H1 * O1 + H1] + td_ref[:, 0:H1]
    l = jnp.where(l > 0, l, 0.2 * l)
    ex_self = jnp.exp(l)
    den = s8 + ex_self + 1e-16
    r8 = r8_ref[...]
    bcast_ex = jnp.dot(ex_self, r8, preferred_element_type=jnp.float32)
    bcast_rd = jnp.dot(1.0 / den, r8, preferred_element_type=jnp.float32)
    out1 = (rawh + bcast_ex * ts[:, 0:H1 * O1]) * bcast_rd
    t = out1 + b1_ref[...]
    h2 = jnp.where(t > 0, t, jnp.exp(jnp.minimum(t, 0.0)) - 1.0)
    ts2_ref[...] = jnp.dot(h2, ws_ref[...],
                           preferred_element_type=jnp.float32)
    td2_ref[...] = jnp.dot(h2, wd_ref[...],
                           preferred_element_type=jnp.float32)


def _dense2(raw1b, ts1, td1, b1, R8, W2ext, Wd2):
    blk = 1000
    return pl.pallas_call(
        _tc2_body,
        grid=(N // blk,),
        in_specs=[
            pl.BlockSpec((blk, 128), lambda i: (i, 0)),
            pl.BlockSpec((blk, 128), lambda i: (i, 0)),
            pl.BlockSpec((blk, 128), lambda i: (i, 0)),
            pl.BlockSpec((blk, 128), lambda i: (i, 0)),
            pl.BlockSpec((blk, 128), lambda i: (i, 0)),
            pl.BlockSpec((blk, W1SRC), lambda i: (i, 0)),
            pl.BlockSpec((blk, 128), lambda i: (i, 0)),
            pl.BlockSpec((1, H1 * O1), lambda i: (0, 0)),
            pl.BlockSpec((H1, H1 * O1), lambda i: (0, 0)),
            pl.BlockSpec((H1 * O1, W2SRC), lambda i: (0, 0)),
            pl.BlockSpec((H1 * O1, 128), lambda i: (0, 0)),
        ],
        out_specs=[
            pl.BlockSpec((blk, W2SRC), lambda i: (i, 0)),
            pl.BlockSpec((blk, 128), lambda i: (i, 0)),
        ],
        out_shape=[
            jax.ShapeDtypeStruct((N, W2SRC), jnp.float32),
            jax.ShapeDtypeStruct((N, 128), jnp.float32),
        ],
    )(raw1b[0], raw1b[1], raw1b[2], raw1b[3], raw1b[4], ts1, td1,
      b1.reshape(1, H1 * O1), R8, W2ext, Wd2)


def _tc3_body(rh_ref, re_ref, ts_ref, td_ref, b2_ref, out_ref):
    ts = ts_ref[...]
    l = ts[:, D:D + 1] + td_ref[:, 0:1]
    l = jnp.where(l > 0, l, 0.2 * l)
    ex = jnp.exp(l)
    den = re_ref[:, 0:1] + ex + 1e-16
    out_ref[...] = ((rh_ref[...] + ex * ts[:, 0:D]) * (1.0 / den)
                    + b2_ref[...])


def _dense3(raw2b, ts2, td2, b2):
    blk = 1000
    return pl.pallas_call(
        _tc3_body,
        grid=(N // blk,),
        in_specs=[
            pl.BlockSpec((blk, 128), lambda i: (i, 0)),
            pl.BlockSpec((blk, 128), lambda i: (i, 0)),
            pl.BlockSpec((blk, W2SRC), lambda i: (i, 0)),
            pl.BlockSpec((blk, 128), lambda i: (i, 0)),
            pl.BlockSpec((1, D), lambda i: (0, 0)),
        ],
        out_specs=pl.BlockSpec((blk, D), lambda i: (i, 0)),
        out_shape=jax.ShapeDtypeStruct((N, D), jnp.float32),
    )(raw2b[0], raw2b[1], ts2, td2, b2.reshape(1, D))


def _head_projector(att, heads, odim):
    eye = jnp.eye(heads, dtype=jnp.float32)
    A = (eye[:, None, :] * att[:, :, None]).reshape(heads * odim, heads)
    return jnp.concatenate(
        [A, jnp.zeros((heads * odim, 16 - heads), jnp.float32)], axis=1)


def kernel(x, edge_index, gamma, beta, W1, att_src1, att_dst1, b1, W2,
           att_src2, att_dst2, b2):
    src = edge_index[0].astype(jnp.int32)
    dst = edge_index[1].astype(jnp.int32)

    A_s1 = _head_projector(att_src1, H1, O1)
    A_d1 = _head_projector(att_dst1, H1, O1)
    W1ext = jnp.concatenate(
        [W1, W1 @ A_s1, jnp.zeros((D, W1SRC - H1 * O1 - 16), jnp.float32)],
        axis=1)
    Wd1 = jnp.concatenate(
        [W1 @ A_d1, jnp.zeros((D, 112), jnp.float32)], axis=1)
    A_s2 = _head_projector(att_src2, 1, D)
    A_d2 = _head_projector(att_dst2, 1, D)
    W2ext = jnp.concatenate(
        [W2, W2 @ A_s2, jnp.zeros((H1 * O1, W2SRC - D - 16), jnp.float32)],
        axis=1)
    Wd2 = jnp.concatenate(
        [W2 @ A_d2, jnp.zeros((H1 * O1, 112), jnp.float32)], axis=1)
    R8 = (jnp.eye(H1, dtype=jnp.float32)[:, None, :]
          * jnp.ones((H1, O1, 1), jnp.float32)).reshape(H1 * O1, H1).T

    ep1 = _make_edge_pass(H1 * O1, H1, HALF // 4)
    ep2 = _make_edge_pass(D, 1, HALF // 2)

    ts1, td1 = _dense1(x, gamma, beta, W1ext, Wd1)
    td1p = jnp.concatenate(
        [td1, jnp.full((1, 128), NEG, jnp.float32)], axis=0)
    zeros = jnp.zeros((16, 128), jnp.float32)
    raw1b = [r[:N] for r in ep1(src, dst, ts1, td1p, zeros)]

    ts2, td2 = _dense2(raw1b, ts1, td1, b1, R8, W2ext, Wd2)
    td2p = jnp.concatenate(
        [td2, jnp.full((1, 128), NEG, jnp.float32)], axis=0)
    raw2b = [r[:N] for r in ep2(src, dst, ts2, td2p, zeros)]

    return _dense3(raw2b, ts2, td2, b2)
